# Initial kernel scaffold; baseline (speedup 1.0000x reference)
import jax
import jax.numpy as jnp
from jax.experimental import pallas as pl


def kernel(atom_features, pair_features, pair_split, atom_to_pair, unused1, unused2, W_AA, b_AA, W_PA, b_PA, W_A, b_A, W_AP, b_AP, W_PP, b_PP, W_P, b_P):
    raise NotImplementedError("write your pallas kernel here")


# speedup vs baseline: 2.5515x; 2.5515x over previous
import jax
import jax.numpy as jnp
from jax import lax
from jax.experimental import pallas as pl
from jax.experimental.pallas import tpu as pltpu
from jax.experimental.pallas import tpu_sc as plsc

N = 50000
E = 800000
HP = 64
NT = 32
HALF = 25000
TROWS = 25600
TSLICE = TROWS // 16
GA = 40
GB = 80
def _sc_weave(x12, ii, jj, ids2, pae):
  mesh = plsc.VectorSubcoreMesh(core_axis_name="c", subcore_axis_name="s")

  def body(x12_h, ii_h, jj_h, ids_h, pae_h, iota_h, zeros_h, s_h, pa_h,
           idx_i, idx_j, buf_a, buf_b, sbuf, idx_b, paebuf, table, sem):
    c = lax.axis_index("c")
    s = lax.axis_index("s")
    w = s * 2 + c

    def phase_a():
      def chunk_a(k, _):
        base = w * (E // NT) + k * GA
        pltpu.sync_copy(ii_h.at[pl.ds(base, GA)], idx_i)
        pltpu.sync_copy(jj_h.at[pl.ds(base, GA)], idx_j)
        pltpu.async_copy(x12_h.at[idx_i], buf_a, sem).wait()
        pltpu.async_copy(x12_h.at[idx_j], buf_b, sem).wait()

        def row(r, _):
          for q in range(HP // 16):
            lo = q * 16
            hi = HP + q * 16
            a1 = buf_a[r, pl.ds(lo, 16)]
            a2 = buf_a[r, pl.ds(hi, 16)]
            b1 = buf_b[r, pl.ds(lo, 16)]
            b2 = buf_b[r, pl.ds(hi, 16)]
            sbuf[r, pl.ds(lo, 16)] = (jnp.maximum(a1 + b2, 0.0)
                                      + jnp.maximum(b1 + a2, 0.0))
          return 0

        lax.fori_loop(0, GA, row, 0)
        pltpu.sync_copy(sbuf, s_h.at[pl.ds(base, GA)])
        return 0

      lax.fori_loop(0, (E // NT) // GA, chunk_a, 0)

    phase_a()

    def phase_b():
      pltpu.sync_copy(zeros_h, paebuf)

      def zchunk(z, _):
        off = s * TSLICE + z * GB
        pltpu.sync_copy(iota_h.at[pl.ds(off, GB)], idx_b)
        pltpu.sync_copy(paebuf, table.at[idx_b])
        return 0

      lax.fori_loop(0, TSLICE // GB, zchunk, 0)
      plsc.subcore_barrier()

      def chunk_b(k, _):
        base = s * (E // 16) + k * GB
        pltpu.sync_copy(ids_h.at[pl.ds(c * E + base, GB)], idx_b)
        pltpu.sync_copy(pae_h.at[pl.ds(base, GB)], paebuf)
        pltpu.sync_copy(paebuf, table.at[idx_b], add=True)
        return 0

      lax.fori_loop(0, (E // 16) // GB, chunk_b, 0)
      plsc.subcore_barrier()

      def out_chunk(k, _):
        off = s * TSLICE + k * GB
        pltpu.sync_copy(iota_h.at[pl.ds(off, GB)], idx_b)
        pltpu.sync_copy(table.at[idx_b], paebuf)
        pltpu.sync_copy(paebuf, pa_h.at[pl.ds(c * TROWS + off, GB)])
        return 0

      lax.fori_loop(0, TSLICE // GB, out_chunk, 0)

    phase_b()

  f = pl.kernel(
      body,
      out_type=(jax.ShapeDtypeStruct((E, HP), jnp.float32),
                jax.ShapeDtypeStruct((2 * TROWS, HP), jnp.float32)),
      mesh=mesh,
      scratch_types=[
          pltpu.VMEM((GA,), jnp.int32),
          pltpu.VMEM((GA,), jnp.int32),
          pltpu.VMEM((GA, 2 * HP), jnp.float32),
          pltpu.VMEM((GA, 2 * HP), jnp.float32),
          pltpu.VMEM((GA, HP), jnp.float32),
          pltpu.VMEM((GB,), jnp.int32),
          pltpu.VMEM((GB, HP), jnp.float32),
          pltpu.VMEM_SHARED((TROWS, HP), jnp.float32),
          pltpu.SemaphoreType.DMA,
      ],
  )
  iota_t = jnp.arange(TROWS, dtype=jnp.int32)
  zeros_t = jnp.zeros((GB, HP), jnp.float32)
  return f(x12, ii, jj, ids2, pae, iota_t, zeros_t)


def _tc_x12(atom, wcat, b1):
  BN = 400

  def body(a_ref, w_ref, b_ref, o_ref):
    o_ref[...] = jnp.dot(a_ref[...], w_ref[...],
                         preferred_element_type=jnp.float32) + b_ref[...]

  return pl.pallas_call(
      body,
      grid=(N // BN,),
      in_specs=[pl.BlockSpec((BN, 75), lambda n: (n, 0)),
                pl.BlockSpec((75, 2 * HP), lambda n: (0, 0)),
                pl.BlockSpec((1, 2 * HP), lambda n: (0, 0))],
      out_specs=pl.BlockSpec((BN, 2 * HP), lambda n: (n, 0)),
      out_shape=jax.ShapeDtypeStruct((N, 2 * HP), jnp.float32),
  )(atom, wcat, b1)


def _tc_pae(pair, w, b):
  BE = 2000

  def body(p_ref, w_ref, b_ref, o_ref):
    o_ref[...] = jnp.maximum(
        jnp.dot(p_ref[...], w_ref[...],
                preferred_element_type=jnp.float32) + b_ref[...], 0.0)

  return pl.pallas_call(
      body,
      grid=(E // BE,),
      in_specs=[pl.BlockSpec((BE, 16), lambda n: (n, 0)),
                pl.BlockSpec((16, HP), lambda n: (0, 0)),
                pl.BlockSpec((1, HP), lambda n: (0, 0))],
      out_specs=pl.BlockSpec((BE, HP), lambda n: (n, 0)),
      out_shape=jax.ShapeDtypeStruct((E, HP), jnp.float32),
  )(pair, w, b)


def _tc_pair_out(pair, s_arr, wpp, bpp, wp1, wp2, bp):
  BE = 2000
  H = 50

  def body(p_ref, s_ref, wpp_ref, bpp_ref, wp1_ref, wp2_ref, bp_ref, o_ref):
    pp = jnp.maximum(
        jnp.dot(p_ref[...], wpp_ref[...],
                preferred_element_type=jnp.float32) + bpp_ref[...], 0.0)
    acc = jnp.dot(s_ref[...], wp1_ref[...], preferred_element_type=jnp.float32)
    acc = acc + jnp.dot(pp, wp2_ref[...], preferred_element_type=jnp.float32)
    o_ref[...] = jnp.maximum(acc + bp_ref[...], 0.0)

  return pl.pallas_call(
      body,
      grid=(E // BE,),
      in_specs=[pl.BlockSpec((BE, 16), lambda n: (n, 0)),
                pl.BlockSpec((BE, HP), lambda n: (n, 0)),
                pl.BlockSpec((16, H), lambda n: (0, 0)),
                pl.BlockSpec((1, H), lambda n: (0, 0)),
                pl.BlockSpec((HP, H), lambda n: (0, 0)),
                pl.BlockSpec((H, H), lambda n: (0, 0)),
                pl.BlockSpec((1, H), lambda n: (0, 0))],
      out_specs=pl.BlockSpec((BE, H), lambda n: (n, 0)),
      out_shape=jax.ShapeDtypeStruct((E, H), jnp.float32),
  )(pair, s_arr, wpp, bpp, wp1, wp2, bp)


def _tc_atom_out(atom, pa, waa, baa, wa1, wa2, ba):
  BN = 200
  H = 50
  blocks_per_half = HALF // BN
  half_stride = TROWS // BN

  def body(a_ref, pa_ref, waa_ref, baa_ref, wa1_ref, wa2_ref, ba_ref, o_ref):
    aa = jnp.maximum(
        jnp.dot(a_ref[...], waa_ref[...],
                preferred_element_type=jnp.float32) + baa_ref[...], 0.0)
    acc = jnp.dot(aa, wa1_ref[...], preferred_element_type=jnp.float32)
    acc = acc + jnp.dot(pa_ref[...], wa2_ref[...],
                        preferred_element_type=jnp.float32)
    o_ref[...] = jnp.maximum(acc + ba_ref[...], 0.0)

  return pl.pallas_call(
      body,
      grid=(N // BN,),
      in_specs=[pl.BlockSpec((BN, 75), lambda n: (n, 0)),
                pl.BlockSpec(
                    (BN, HP),
                    lambda n: (half_stride * (n // blocks_per_half)
                               + n % blocks_per_half, 0)),
                pl.BlockSpec((75, H), lambda n: (0, 0)),
                pl.BlockSpec((1, H), lambda n: (0, 0)),
                pl.BlockSpec((H, H), lambda n: (0, 0)),
                pl.BlockSpec((HP, H), lambda n: (0, 0)),
                pl.BlockSpec((1, H), lambda n: (0, 0))],
      out_specs=pl.BlockSpec((BN, H), lambda n: (n, 0)),
      out_shape=jax.ShapeDtypeStruct((N, H), jnp.float32),
  )(atom, pa, waa, baa, wa1, wa2, ba)


def kernel(atom_features, pair_features, pair_split, atom_to_pair,
           unused1, unused2, W_AA, b_AA, W_PA, b_PA, W_A, b_A,
           W_AP, b_AP, W_PP, b_PP, W_P, b_P):
  f32 = jnp.float32
  D_A = atom_features.shape[1]
  H = W_AA.shape[1]

  w1 = jnp.zeros((D_A, HP), f32).at[:, :H].set(W_AP[:D_A])
  w2 = jnp.zeros((D_A, HP), f32).at[:, :H].set(W_AP[D_A:])
  wcat = jnp.concatenate([w1, w2], axis=1)
  b1 = jnp.zeros((1, 2 * HP), f32).at[0, :H].set(b_AP)
  wpa = jnp.zeros((W_PA.shape[0], HP), f32).at[:, :H].set(W_PA)
  bpa = jnp.zeros((1, HP), f32).at[0, :H].set(b_PA)
  wp1 = jnp.zeros((HP, H), f32).at[:H].set(W_P[:H])
  wa2 = jnp.zeros((HP, H), f32).at[:H].set(W_A[H:])

  ii = atom_to_pair[:, 0].astype(jnp.int32)
  jj = atom_to_pair[:, 1].astype(jnp.int32)
  ids = pair_split.astype(jnp.int32)

  x12 = _tc_x12(atom_features, wcat, b1)
  pae = _tc_pae(pair_features, wpa, bpa)
  ids2 = jnp.stack([ids, ids - HALF])
  ids2 = jnp.where((ids2 >= 0) & (ids2 < HALF), ids2, HALF).reshape(2 * E)
  s_arr, pa_arr = _sc_weave(x12, ii, jj, ids2, pae)

  P = _tc_pair_out(pair_features, s_arr, W_PP, b_PP[None], wp1,
                   W_P[H:], b_P[None])
  A = _tc_atom_out(atom_features, pa_arr, W_AA, b_AA[None], W_A[:H],
                   wa2, b_A[None])
  return (A, P)


# speedup vs baseline: 2.7767x; 1.0883x over previous
import jax
import jax.numpy as jnp
from jax import lax
from jax.experimental import pallas as pl
from jax.experimental.pallas import tpu as pltpu
from jax.experimental.pallas import tpu_sc as plsc

N = 50000
E = 800000
HP = 64
NT = 32
HALF = 25000
TROWS = 25600
TSLICE = TROWS // 16
WB = 56
GA = 40
GB = 40
def _sc_weave(x12, ii, jj, ids2, pae):
  mesh = plsc.VectorSubcoreMesh(core_axis_name="c", subcore_axis_name="s")

  def body(x12_h, ii_h, jj_h, ids_h, pae_h, iota_h, zeros_h, s_h, pa_h,
           idx_i0, idx_j0, buf_a0, buf_b0, idx_i1, idx_j1, buf_a1, buf_b1,
           sbuf, idx_b, paebuf, table, sem_a, sem_b):
    c = lax.axis_index("c")
    s = lax.axis_index("s")
    w = s * 2 + c
    ebase = w * (E // NT)

    def fire(kv, idx_i, idx_j, buf_a, buf_b, sem):
      base = ebase + kv * GA
      pltpu.sync_copy(ii_h.at[pl.ds(base, GA)], idx_i)
      pltpu.sync_copy(jj_h.at[pl.ds(base, GA)], idx_j)
      pltpu.async_copy(x12_h.at[idx_i], buf_a, sem)
      pltpu.async_copy(x12_h.at[idx_j], buf_b, sem)

    def drain(idx_i, idx_j, buf_a, buf_b, sem):
      pltpu.make_async_copy(x12_h.at[idx_i], buf_a, sem).wait()
      pltpu.make_async_copy(x12_h.at[idx_j], buf_b, sem).wait()

    def compute_store(kv, buf_a, buf_b):
      def row(r, _):
        for q in range(HP // 16):
          lo = q * 16
          hi = HP + q * 16
          a1 = buf_a[r, pl.ds(lo, 16)]
          a2 = buf_a[r, pl.ds(hi, 16)]
          b1 = buf_b[r, pl.ds(lo, 16)]
          b2 = buf_b[r, pl.ds(hi, 16)]
          sbuf[r, pl.ds(lo, 16)] = (jnp.maximum(a1 + b2, 0.0)
                                    + jnp.maximum(b1 + a2, 0.0))
        return 0

      lax.fori_loop(0, GA, row, 0)
      pltpu.sync_copy(sbuf, s_h.at[pl.ds(ebase + kv * GA, GA)])

    nch = (E // NT) // GA
    fire(0, idx_i0, idx_j0, buf_a0, buf_b0, sem_a)

    def chunk_pair(t, _):
      fire(2 * t + 1, idx_i1, idx_j1, buf_a1, buf_b1, sem_b)
      drain(idx_i0, idx_j0, buf_a0, buf_b0, sem_a)
      compute_store(2 * t, buf_a0, buf_b0)
      fire(2 * t + 2, idx_i0, idx_j0, buf_a0, buf_b0, sem_a)
      drain(idx_i1, idx_j1, buf_a1, buf_b1, sem_b)
      compute_store(2 * t + 1, buf_a1, buf_b1)
      return 0

    lax.fori_loop(0, nch // 2, chunk_pair, 0)
    drain(idx_i0, idx_j0, buf_a0, buf_b0, sem_a)
    compute_store(nch - 1, buf_a0, buf_b0)

    def phase_b():
      pltpu.sync_copy(zeros_h, paebuf)

      def zchunk(z, _):
        off = s * TSLICE + z * GB
        pltpu.sync_copy(iota_h.at[pl.ds(off, GB)], idx_b)
        pltpu.sync_copy(paebuf, table.at[idx_b])
        return 0

      lax.fori_loop(0, TSLICE // GB, zchunk, 0)
      plsc.subcore_barrier()

      def chunk_b(k, _):
        base = s * (E // 16) + k * GB
        pltpu.sync_copy(ids_h.at[pl.ds(c * E + base, GB)], idx_b)
        pltpu.sync_copy(pae_h.at[pl.ds(base, GB)], paebuf)
        pltpu.sync_copy(paebuf, table.at[idx_b], add=True)
        return 0

      lax.fori_loop(0, (E // 16) // GB, chunk_b, 0)
      plsc.subcore_barrier()

      def out_chunk(k, _):
        off = s * TSLICE + k * GB
        pltpu.sync_copy(iota_h.at[pl.ds(off, GB)], idx_b)
        pltpu.sync_copy(table.at[idx_b], paebuf)
        pltpu.sync_copy(paebuf, pa_h.at[pl.ds(c * TROWS + off, GB)])
        return 0

      lax.fori_loop(0, TSLICE // GB, out_chunk, 0)

    phase_b()

  f = pl.kernel(
      body,
      out_type=(jax.ShapeDtypeStruct((E, HP), jnp.float32),
                jax.ShapeDtypeStruct((2 * TROWS, WB), jnp.float32)),
      mesh=mesh,
      scratch_types=[
          pltpu.VMEM((GA,), jnp.int32),
          pltpu.VMEM((GA,), jnp.int32),
          pltpu.VMEM((GA, 2 * HP), jnp.float32),
          pltpu.VMEM((GA, 2 * HP), jnp.float32),
          pltpu.VMEM((GA,), jnp.int32),
          pltpu.VMEM((GA,), jnp.int32),
          pltpu.VMEM((GA, 2 * HP), jnp.float32),
          pltpu.VMEM((GA, 2 * HP), jnp.float32),
          pltpu.VMEM((GA, HP), jnp.float32),
          pltpu.VMEM((GB,), jnp.int32),
          pltpu.VMEM((GB, WB), jnp.float32),
          pltpu.VMEM_SHARED((TROWS, WB), jnp.float32),
          pltpu.SemaphoreType.DMA,
          pltpu.SemaphoreType.DMA,
      ],
  )
  iota_t = jnp.arange(TROWS, dtype=jnp.int32)
  zeros_t = jnp.zeros((GB, WB), jnp.float32)
  return f(x12, ii, jj, ids2, pae, iota_t, zeros_t)


def _tc_x12(atom, wcat, b1):
  BN = 400

  def body(a_ref, w_ref, b_ref, o_ref):
    o_ref[...] = jnp.dot(a_ref[...], w_ref[...],
                         preferred_element_type=jnp.float32) + b_ref[...]

  return pl.pallas_call(
      body,
      grid=(N // BN,),
      in_specs=[pl.BlockSpec((BN, 75), lambda n: (n, 0)),
                pl.BlockSpec((75, 2 * HP), lambda n: (0, 0)),
                pl.BlockSpec((1, 2 * HP), lambda n: (0, 0))],
      out_specs=pl.BlockSpec((BN, 2 * HP), lambda n: (n, 0)),
      out_shape=jax.ShapeDtypeStruct((N, 2 * HP), jnp.float32),
  )(atom, wcat, b1)


def _tc_pae(pair, w, b):
  BE = 2000

  def body(p_ref, w_ref, b_ref, o_ref):
    o_ref[...] = jnp.maximum(
        jnp.dot(p_ref[...], w_ref[...],
                preferred_element_type=jnp.float32) + b_ref[...], 0.0)

  return pl.pallas_call(
      body,
      grid=(E // BE,),
      in_specs=[pl.BlockSpec((BE, 16), lambda n: (n, 0)),
                pl.BlockSpec((16, WB), lambda n: (0, 0)),
                pl.BlockSpec((1, WB), lambda n: (0, 0))],
      out_specs=pl.BlockSpec((BE, WB), lambda n: (n, 0)),
      out_shape=jax.ShapeDtypeStruct((E, WB), jnp.float32),
  )(pair, w, b)


def _tc_pair_out(pair, s_arr, wpp, bpp, wp1, wp2, bp):
  BE = 2000
  H = 50

  def body(p_ref, s_ref, wpp_ref, bpp_ref, wp1_ref, wp2_ref, bp_ref, o_ref):
    pp = jnp.maximum(
        jnp.dot(p_ref[...], wpp_ref[...],
                preferred_element_type=jnp.float32) + bpp_ref[...], 0.0)
    acc = jnp.dot(s_ref[...], wp1_ref[...], preferred_element_type=jnp.float32)
    acc = acc + jnp.dot(pp, wp2_ref[...], preferred_element_type=jnp.float32)
    o_ref[...] = jnp.maximum(acc + bp_ref[...], 0.0)

  return pl.pallas_call(
      body,
      grid=(E // BE,),
      in_specs=[pl.BlockSpec((BE, 16), lambda n: (n, 0)),
                pl.BlockSpec((BE, HP), lambda n: (n, 0)),
                pl.BlockSpec((16, H), lambda n: (0, 0)),
                pl.BlockSpec((1, H), lambda n: (0, 0)),
                pl.BlockSpec((HP, H), lambda n: (0, 0)),
                pl.BlockSpec((H, H), lambda n: (0, 0)),
                pl.BlockSpec((1, H), lambda n: (0, 0))],
      out_specs=pl.BlockSpec((BE, H), lambda n: (n, 0)),
      out_shape=jax.ShapeDtypeStruct((E, H), jnp.float32),
  )(pair, s_arr, wpp, bpp, wp1, wp2, bp)


def _tc_atom_out(atom, pa, waa, baa, wa1, wa2, ba):
  BN = 200
  H = 50
  blocks_per_half = HALF // BN
  half_stride = TROWS // BN

  def body(a_ref, pa_ref, waa_ref, baa_ref, wa1_ref, wa2_ref, ba_ref, o_ref):
    aa = jnp.maximum(
        jnp.dot(a_ref[...], waa_ref[...],
                preferred_element_type=jnp.float32) + baa_ref[...], 0.0)
    acc = jnp.dot(aa, wa1_ref[...], preferred_element_type=jnp.float32)
    acc = acc + jnp.dot(pa_ref[...], wa2_ref[...],
                        preferred_element_type=jnp.float32)
    o_ref[...] = jnp.maximum(acc + ba_ref[...], 0.0)

  return pl.pallas_call(
      body,
      grid=(N // BN,),
      in_specs=[pl.BlockSpec((BN, 75), lambda n: (n, 0)),
                pl.BlockSpec(
                    (BN, WB),
                    lambda n: (half_stride * (n // blocks_per_half)
                               + n % blocks_per_half, 0)),
                pl.BlockSpec((75, H), lambda n: (0, 0)),
                pl.BlockSpec((1, H), lambda n: (0, 0)),
                pl.BlockSpec((H, H), lambda n: (0, 0)),
                pl.BlockSpec((WB, H), lambda n: (0, 0)),
                pl.BlockSpec((1, H), lambda n: (0, 0))],
      out_specs=pl.BlockSpec((BN, H), lambda n: (n, 0)),
      out_shape=jax.ShapeDtypeStruct((N, H), jnp.float32),
  )(atom, pa, waa, baa, wa1, wa2, ba)


def kernel(atom_features, pair_features, pair_split, atom_to_pair,
           unused1, unused2, W_AA, b_AA, W_PA, b_PA, W_A, b_A,
           W_AP, b_AP, W_PP, b_PP, W_P, b_P):
  f32 = jnp.float32
  D_A = atom_features.shape[1]
  H = W_AA.shape[1]

  w1 = jnp.zeros((D_A, HP), f32).at[:, :H].set(W_AP[:D_A])
  w2 = jnp.zeros((D_A, HP), f32).at[:, :H].set(W_AP[D_A:])
  wcat = jnp.concatenate([w1, w2], axis=1)
  b1 = jnp.zeros((1, 2 * HP), f32).at[0, :H].set(b_AP)
  wpa = jnp.zeros((W_PA.shape[0], WB), f32).at[:, :H].set(W_PA)
  bpa = jnp.zeros((1, WB), f32).at[0, :H].set(b_PA)
  wp1 = jnp.zeros((HP, H), f32).at[:H].set(W_P[:H])
  wa2 = jnp.zeros((WB, H), f32).at[:H].set(W_A[H:])

  ii = atom_to_pair[:, 0].astype(jnp.int32)
  jj = atom_to_pair[:, 1].astype(jnp.int32)
  ids = pair_split.astype(jnp.int32)

  x12 = _tc_x12(atom_features, wcat, b1)
  pae = _tc_pae(pair_features, wpa, bpa)
  ids2 = jnp.stack([ids, ids - HALF])
  ids2 = jnp.where((ids2 >= 0) & (ids2 < HALF), ids2, HALF).reshape(2 * E)
  s_arr, pa_arr = _sc_weave(x12, ii, jj, ids2, pae)

  P = _tc_pair_out(pair_features, s_arr, W_PP, b_PP[None], wp1,
                   W_P[H:], b_P[None])
  A = _tc_atom_out(atom_features, pa_arr, W_AA, b_AA[None], W_A[:H],
                   wa2, b_A[None])
  return (A, P)


# speedup vs baseline: 3.3896x; 1.2207x over previous
import jax
import jax.numpy as jnp
from jax import lax
from jax.experimental import pallas as pl
from jax.experimental.pallas import tpu as pltpu
from jax.experimental.pallas import tpu_sc as plsc

N = 50000
E = 800000
HP = 64
NT = 32
HALF = 25000
TROWS = 25600
TSLICE = TROWS // 16
WB = 56
GA = 40
GB = 40
def _sc_weave(x12, ii, jj, ids2, pae):
  mesh = plsc.VectorSubcoreMesh(core_axis_name="c", subcore_axis_name="s")

  def body(x12_h, ii_h, jj_h, ids_h, pae_h, iota_h, zeros_h, s_h, pa_h,
           idx_i0, idx_j0, buf_a0, buf_b0, idx_i1, idx_j1, buf_a1, buf_b1,
           sbuf, idx_b, paebuf, idx_b1, paebuf1, table, sem_a, sem_b):
    c = lax.axis_index("c")
    s = lax.axis_index("s")
    w = s * 2 + c
    ebase = w * (E // NT)

    def fire(kv, idx_i, idx_j, buf_a, buf_b, sem):
      base = ebase + kv * GA
      pltpu.sync_copy(ii_h.at[pl.ds(base, GA)], idx_i)
      pltpu.sync_copy(jj_h.at[pl.ds(base, GA)], idx_j)
      pltpu.async_copy(x12_h.at[idx_i], buf_a, sem)
      pltpu.async_copy(x12_h.at[idx_j], buf_b, sem)

    def drain(idx_i, idx_j, buf_a, buf_b, sem):
      pltpu.make_async_copy(x12_h.at[idx_i], buf_a, sem).wait()
      pltpu.make_async_copy(x12_h.at[idx_j], buf_b, sem).wait()

    def compute_store(kv, buf_a, buf_b):
      def row(r, _):
        for q in range(HP // 16):
          lo = q * 16
          hi = HP + q * 16
          a1 = buf_a[r, pl.ds(lo, 16)]
          a2 = buf_a[r, pl.ds(hi, 16)]
          b1 = buf_b[r, pl.ds(lo, 16)]
          b2 = buf_b[r, pl.ds(hi, 16)]
          sbuf[r, pl.ds(lo, 16)] = (jnp.maximum(a1 + b2, 0.0)
                                    + jnp.maximum(b1 + a2, 0.0))
        return 0

      lax.fori_loop(0, GA, row, 0)
      pltpu.sync_copy(sbuf, s_h.at[pl.ds(ebase + kv * GA, GA)])

    nch = (E // NT) // GA
    fire(0, idx_i0, idx_j0, buf_a0, buf_b0, sem_a)

    def chunk_pair(t, _):
      fire(2 * t + 1, idx_i1, idx_j1, buf_a1, buf_b1, sem_b)
      drain(idx_i0, idx_j0, buf_a0, buf_b0, sem_a)
      compute_store(2 * t, buf_a0, buf_b0)
      fire(2 * t + 2, idx_i0, idx_j0, buf_a0, buf_b0, sem_a)
      drain(idx_i1, idx_j1, buf_a1, buf_b1, sem_b)
      compute_store(2 * t + 1, buf_a1, buf_b1)
      return 0

    lax.fori_loop(0, nch // 2, chunk_pair, 0)
    drain(idx_i0, idx_j0, buf_a0, buf_b0, sem_a)
    compute_store(nch - 1, buf_a0, buf_b0)

    def phase_b():
      pltpu.sync_copy(zeros_h, paebuf)

      def zchunk(z, _):
        off = s * TSLICE + z * GB
        pltpu.sync_copy(iota_h.at[pl.ds(off, GB)], idx_b)
        pltpu.sync_copy(paebuf, table.at[idx_b])
        return 0

      lax.fori_loop(0, TSLICE // GB, zchunk, 0)
      plsc.subcore_barrier()

      nchb = (E // 16) // GB

      def fire_b(kv, idxb, buf, sem):
        @pl.when(kv < nchb)
        def _():
          base = s * (E // 16) + kv * GB
          pltpu.sync_copy(ids_h.at[pl.ds(c * E + base, GB)], idxb)
          pltpu.async_copy(pae_h.at[pl.ds(base, GB)], buf, sem)

      def scatter_b(idxb, buf, sem):
        pltpu.make_async_copy(pae_h.at[pl.ds(0, GB)], buf, sem).wait()
        pltpu.sync_copy(buf, table.at[idxb], add=True)

      fire_b(0, idx_b, paebuf, sem_a)

      def pair_b(t, _):
        fire_b(2 * t + 1, idx_b1, paebuf1, sem_b)
        scatter_b(idx_b, paebuf, sem_a)
        fire_b(2 * t + 2, idx_b, paebuf, sem_a)
        scatter_b(idx_b1, paebuf1, sem_b)
        return 0

      lax.fori_loop(0, nchb // 2, pair_b, 0)
      plsc.subcore_barrier()

      def out_chunk(k, _):
        off = s * TSLICE + k * GB
        pltpu.sync_copy(iota_h.at[pl.ds(off, GB)], idx_b)
        pltpu.sync_copy(table.at[idx_b], paebuf)
        pltpu.sync_copy(paebuf, pa_h.at[pl.ds(c * TROWS + off, GB)])
        return 0

      lax.fori_loop(0, TSLICE // GB, out_chunk, 0)

    phase_b()

  f = pl.kernel(
      body,
      out_type=(jax.ShapeDtypeStruct((E, HP), jnp.float32),
                jax.ShapeDtypeStruct((2 * TROWS, WB), jnp.float32)),
      mesh=mesh,
      scratch_types=[
          pltpu.VMEM((GA,), jnp.int32),
          pltpu.VMEM((GA,), jnp.int32),
          pltpu.VMEM((GA, 2 * HP), jnp.float32),
          pltpu.VMEM((GA, 2 * HP), jnp.float32),
          pltpu.VMEM((GA,), jnp.int32),
          pltpu.VMEM((GA,), jnp.int32),
          pltpu.VMEM((GA, 2 * HP), jnp.float32),
          pltpu.VMEM((GA, 2 * HP), jnp.float32),
          pltpu.VMEM((GA, HP), jnp.float32),
          pltpu.VMEM((GB,), jnp.int32),
          pltpu.VMEM((GB, WB), jnp.float32),
          pltpu.VMEM((GB,), jnp.int32),
          pltpu.VMEM((GB, WB), jnp.float32),
          pltpu.VMEM_SHARED((TROWS, WB), jnp.float32),
          pltpu.SemaphoreType.DMA,
          pltpu.SemaphoreType.DMA,
      ],
  )
  iota_t = jnp.arange(TROWS, dtype=jnp.int32)
  zeros_t = jnp.zeros((GB, WB), jnp.float32)
  return f(x12, ii, jj, ids2, pae, iota_t, zeros_t)


def _tc_x12(atom, wcat, b1):
  BN = 400

  def body(a_ref, w_ref, b_ref, o_ref):
    o_ref[...] = jnp.dot(a_ref[...], w_ref[...],
                         preferred_element_type=jnp.float32) + b_ref[...]

  return pl.pallas_call(
      body,
      grid=(N // BN,),
      in_specs=[pl.BlockSpec((BN, 75), lambda n: (n, 0)),
                pl.BlockSpec((75, 2 * HP), lambda n: (0, 0)),
                pl.BlockSpec((1, 2 * HP), lambda n: (0, 0))],
      out_specs=pl.BlockSpec((BN, 2 * HP), lambda n: (n, 0)),
      out_shape=jax.ShapeDtypeStruct((N, 2 * HP), jnp.float32),
  )(atom, wcat, b1)


def _tc_pae(pair, w, b):
  BE = 2000

  def body(p_ref, w_ref, b_ref, o_ref):
    o_ref[...] = jnp.maximum(
        jnp.dot(p_ref[...], w_ref[...],
                preferred_element_type=jnp.float32) + b_ref[...], 0.0)

  return pl.pallas_call(
      body,
      grid=(E // BE,),
      in_specs=[pl.BlockSpec((BE, 16), lambda n: (n, 0)),
                pl.BlockSpec((16, WB), lambda n: (0, 0)),
                pl.BlockSpec((1, WB), lambda n: (0, 0))],
      out_specs=pl.BlockSpec((BE, WB), lambda n: (n, 0)),
      out_shape=jax.ShapeDtypeStruct((E, WB), jnp.float32),
  )(pair, w, b)


def _tc_pair_out(pair, s_arr, wpp, bpp, wp1, wp2, bp):
  BE = 2000
  H = 50

  def body(p_ref, s_ref, wpp_ref, bpp_ref, wp1_ref, wp2_ref, bp_ref, o_ref):
    pp = jnp.maximum(
        jnp.dot(p_ref[...], wpp_ref[...],
                preferred_element_type=jnp.float32) + bpp_ref[...], 0.0)
    acc = jnp.dot(s_ref[...], wp1_ref[...], preferred_element_type=jnp.float32)
    acc = acc + jnp.dot(pp, wp2_ref[...], preferred_element_type=jnp.float32)
    o_ref[...] = jnp.maximum(acc + bp_ref[...], 0.0)

  return pl.pallas_call(
      body,
      grid=(E // BE,),
      in_specs=[pl.BlockSpec((BE, 16), lambda n: (n, 0)),
                pl.BlockSpec((BE, HP), lambda n: (n, 0)),
                pl.BlockSpec((16, H), lambda n: (0, 0)),
                pl.BlockSpec((1, H), lambda n: (0, 0)),
                pl.BlockSpec((HP, H), lambda n: (0, 0)),
                pl.BlockSpec((H, H), lambda n: (0, 0)),
                pl.BlockSpec((1, H), lambda n: (0, 0))],
      out_specs=pl.BlockSpec((BE, H), lambda n: (n, 0)),
      out_shape=jax.ShapeDtypeStruct((E, H), jnp.float32),
  )(pair, s_arr, wpp, bpp, wp1, wp2, bp)


def _tc_atom_out(atom, pa, waa, baa, wa1, wa2, ba):
  BN = 200
  H = 50
  blocks_per_half = HALF // BN
  half_stride = TROWS // BN

  def body(a_ref, pa_ref, waa_ref, baa_ref, wa1_ref, wa2_ref, ba_ref, o_ref):
    aa = jnp.maximum(
        jnp.dot(a_ref[...], waa_ref[...],
                preferred_element_type=jnp.float32) + baa_ref[...], 0.0)
    acc = jnp.dot(aa, wa1_ref[...], preferred_element_type=jnp.float32)
    acc = acc + jnp.dot(pa_ref[...], wa2_ref[...],
                        preferred_element_type=jnp.float32)
    o_ref[...] = jnp.maximum(acc + ba_ref[...], 0.0)

  return pl.pallas_call(
      body,
      grid=(N // BN,),
      in_specs=[pl.BlockSpec((BN, 75), lambda n: (n, 0)),
                pl.BlockSpec(
                    (BN, WB),
                    lambda n: (half_stride * (n // blocks_per_half)
                               + n % blocks_per_half, 0)),
                pl.BlockSpec((75, H), lambda n: (0, 0)),
                pl.BlockSpec((1, H), lambda n: (0, 0)),
                pl.BlockSpec((H, H), lambda n: (0, 0)),
                pl.BlockSpec((WB, H), lambda n: (0, 0)),
                pl.BlockSpec((1, H), lambda n: (0, 0))],
      out_specs=pl.BlockSpec((BN, H), lambda n: (n, 0)),
      out_shape=jax.ShapeDtypeStruct((N, H), jnp.float32),
  )(atom, pa, waa, baa, wa1, wa2, ba)


def kernel(atom_features, pair_features, pair_split, atom_to_pair,
           unused1, unused2, W_AA, b_AA, W_PA, b_PA, W_A, b_A,
           W_AP, b_AP, W_PP, b_PP, W_P, b_P):
  f32 = jnp.float32
  D_A = atom_features.shape[1]
  H = W_AA.shape[1]

  w1 = jnp.zeros((D_A, HP), f32).at[:, :H].set(W_AP[:D_A])
  w2 = jnp.zeros((D_A, HP), f32).at[:, :H].set(W_AP[D_A:])
  wcat = jnp.concatenate([w1, w2], axis=1)
  b1 = jnp.zeros((1, 2 * HP), f32).at[0, :H].set(b_AP)
  wpa = jnp.zeros((W_PA.shape[0], WB), f32).at[:, :H].set(W_PA)
  bpa = jnp.zeros((1, WB), f32).at[0, :H].set(b_PA)
  wp1 = jnp.zeros((HP, H), f32).at[:H].set(W_P[:H])
  wa2 = jnp.zeros((WB, H), f32).at[:H].set(W_A[H:])

  ii = atom_to_pair[:, 0].astype(jnp.int32)
  jj = atom_to_pair[:, 1].astype(jnp.int32)
  ids = pair_split.astype(jnp.int32)

  x12 = _tc_x12(atom_features, wcat, b1)
  pae = _tc_pae(pair_features, wpa, bpa)
  ids2 = jnp.stack([ids, ids - HALF])
  ids2 = jnp.where((ids2 >= 0) & (ids2 < HALF), ids2, HALF).reshape(2 * E)
  s_arr, pa_arr = _sc_weave(x12, ii, jj, ids2, pae)

  P = _tc_pair_out(pair_features, s_arr, W_PP, b_PP[None], wp1,
                   W_P[H:], b_P[None])
  A = _tc_atom_out(atom_features, pa_arr, W_AA, b_AA[None], W_A[:H],
                   wa2, b_A[None])
  return (A, P)
